# Initial kernel scaffold; baseline (speedup 1.0000x reference)
import jax
import jax.numpy as jnp
from jax.experimental import pallas as pl


def kernel(q_len, k_len, relative_attention_bias):
    raise NotImplementedError("write your pallas kernel here")


# speedup vs baseline: 42.0835x; 42.0835x over previous
import functools
import math

import jax
import jax.numpy as jnp
from jax import lax
from jax.experimental import pallas as pl
from jax.experimental.pallas import tpu as pltpu
from jax.experimental.pallas import tpu_sc as plsc

NUM_BUCKETS = 32
NUM_HEADS = 16
MAX_DISTANCE = 128
Q_LEN = 2048
K_LEN = 2048
DIAG = Q_LEN + K_LEN
NSHIFT = 8
NBUF = 16


def _tc_diag_body(w_s, out_ref):
    h = pl.program_id(0)
    jc = lax.broadcasted_iota(jnp.int32, (1, NSHIFT, DIAG), 2)
    jr = lax.broadcasted_iota(jnp.int32, (1, NSHIFT, DIAG), 1)
    j = jc + jr
    rel = j - (K_LEN - 1)
    num_buckets = NUM_BUCKETS // 2
    n = -rel
    is_neg = n < 0
    n = jnp.abs(n)
    max_exact = num_buckets // 2
    is_small = n < max_exact
    n_clipped = jnp.maximum(n, 1)
    val_if_large = max_exact + (
        jnp.log(n_clipped.astype(jnp.float32) / max_exact)
        / math.log(MAX_DISTANCE / max_exact)
        * (num_buckets - max_exact)
    ).astype(jnp.int32)
    val_if_large = jnp.minimum(val_if_large, num_buckets - 1)
    bucket = jnp.where(is_small, n, val_if_large)
    bucket = jnp.where(is_neg, bucket + num_buckets, bucket)
    acc = jnp.zeros((1, NSHIFT, DIAG), jnp.float32)
    for b in range(NUM_BUCKETS):
        acc = jnp.where(bucket == b, w_s[b, h], acc)
    out_ref[...] = acc


def _sc_body(diag_hbm, out_hbm, dvec8, sem_out):
    c = lax.axis_index("c")
    s = lax.axis_index("s")
    h = s
    qbase = c * (Q_LEN // 2)

    pltpu.sync_copy(diag_hbm.at[pl.ds(h * (NSHIFT * DIAG), NSHIFT * DIAG)], dvec8)

    def row_step(i, carry):
        q = qbase + i
        off = (K_LEN - 1) - q
        r = lax.rem(off, NSHIFT)
        start = pl.multiple_of((off // NSHIFT + r * (DIAG // NSHIFT)) * NSHIFT, NSHIFT)
        pltpu.async_copy(
            dvec8.at[pl.ds(start, K_LEN)],
            out_hbm.at[pl.ds((h * Q_LEN + q) * K_LEN, K_LEN)],
            sem_out,
        )

        @pl.when(i >= NBUF)
        def _wait_one():
            pltpu.make_async_copy(
                dvec8.at[pl.ds(0, K_LEN)],
                out_hbm.at[pl.ds((h * Q_LEN + qbase) * K_LEN, K_LEN)],
                sem_out,
            ).wait()

        return carry

    lax.fori_loop(0, Q_LEN // 2, row_step, 0)

    def drain_step(i, carry):
        pltpu.make_async_copy(
            dvec8.at[pl.ds(0, K_LEN)],
            out_hbm.at[pl.ds((h * Q_LEN + qbase) * K_LEN, K_LEN)],
            sem_out,
        ).wait()
        return carry

    lax.fori_loop(0, NBUF, drain_step, 0)


def kernel(q_len, k_len, relative_attention_bias):
    diag8 = pl.pallas_call(
        _tc_diag_body,
        grid=(NUM_HEADS,),
        in_specs=[pl.BlockSpec(memory_space=pltpu.SMEM)],
        out_specs=pl.BlockSpec((1, NSHIFT, DIAG), lambda i: (i, 0, 0)),
        out_shape=jax.ShapeDtypeStruct((NUM_HEADS, NSHIFT, DIAG), jnp.float32),
    )(relative_attention_bias)

    mesh = plsc.VectorSubcoreMesh(core_axis_name="c", subcore_axis_name="s")
    run = functools.partial(
        pl.kernel,
        mesh=mesh,
        out_type=jax.ShapeDtypeStruct((NUM_HEADS * Q_LEN * K_LEN,), jnp.float32),
        scratch_types=[
            pltpu.VMEM((NSHIFT * DIAG,), jnp.float32),
            pltpu.SemaphoreType.DMA,
        ],
    )(_sc_body)
    out = run(diag8.reshape(NUM_HEADS * NSHIFT * DIAG))
    return out.reshape(1, NUM_HEADS, Q_LEN, K_LEN)


# speedup vs baseline: 42.1702x; 1.0021x over previous
import functools
import math

import jax
import jax.numpy as jnp
from jax import lax
from jax.experimental import pallas as pl
from jax.experimental.pallas import tpu as pltpu
from jax.experimental.pallas import tpu_sc as plsc

NUM_BUCKETS = 32
NUM_HEADS = 16
MAX_DISTANCE = 128
Q_LEN = 2048
K_LEN = 2048
DIAG = Q_LEN + K_LEN
NSHIFT = 8
NGRP = 8


def _tc_diag_body(w_s, out_ref):
    h = pl.program_id(0)
    jc = lax.broadcasted_iota(jnp.int32, (1, NSHIFT, DIAG), 2)
    jr = lax.broadcasted_iota(jnp.int32, (1, NSHIFT, DIAG), 1)
    j = jc + jr
    rel = j - (K_LEN - 1)
    num_buckets = NUM_BUCKETS // 2
    n = -rel
    is_neg = n < 0
    n = jnp.abs(n)
    max_exact = num_buckets // 2
    is_small = n < max_exact
    n_clipped = jnp.maximum(n, 1)
    val_if_large = max_exact + (
        jnp.log(n_clipped.astype(jnp.float32) / max_exact)
        / math.log(MAX_DISTANCE / max_exact)
        * (num_buckets - max_exact)
    ).astype(jnp.int32)
    val_if_large = jnp.minimum(val_if_large, num_buckets - 1)
    bucket = jnp.where(is_small, n, val_if_large)
    bucket = jnp.where(is_neg, bucket + num_buckets, bucket)
    acc = jnp.zeros((1, NSHIFT, DIAG), jnp.float32)
    for b in range(NUM_BUCKETS):
        acc = jnp.where(bucket == b, w_s[b, h], acc)
    out_ref[...] = acc


def _sc_body(diag_hbm, out_hbm, dvec8, sem_out):
    c = lax.axis_index("c")
    s = lax.axis_index("s")
    h = s
    qbase = c * (Q_LEN // 2)

    pltpu.sync_copy(diag_hbm.at[pl.ds(h * (NSHIFT * DIAG), NSHIFT * DIAG)], dvec8)

    def _wait_group():
        pltpu.make_async_copy(
            dvec8.at[pl.ds(0, 8 * K_LEN)],
            out_hbm.at[pl.ds(0, 8 * K_LEN)],
            sem_out,
        ).wait()

    def grp_step(g, carry):
        b8 = pl.multiple_of((255 - c * 128 - g) * 8, 8)
        qrow = pl.multiple_of((h * Q_LEN + qbase + g * 8) * K_LEN, 8)
        for j in range(8):
            pltpu.async_copy(
                dvec8.at[pl.ds(b8 + (7 - j) * DIAG, K_LEN)],
                out_hbm.at[pl.ds(qrow + j * K_LEN, K_LEN)],
                sem_out,
            )

        @pl.when(g >= NGRP)
        def _wait_one():
            _wait_group()

        return carry

    lax.fori_loop(0, Q_LEN // 2 // 8, grp_step, 0)

    def drain_step(i, carry):
        _wait_group()
        return carry

    lax.fori_loop(0, NGRP, drain_step, 0)


def kernel(q_len, k_len, relative_attention_bias):
    diag8 = pl.pallas_call(
        _tc_diag_body,
        grid=(NUM_HEADS,),
        in_specs=[pl.BlockSpec(memory_space=pltpu.SMEM)],
        out_specs=pl.BlockSpec((1, NSHIFT, DIAG), lambda i: (i, 0, 0)),
        out_shape=jax.ShapeDtypeStruct((NUM_HEADS, NSHIFT, DIAG), jnp.float32),
    )(relative_attention_bias)

    mesh = plsc.VectorSubcoreMesh(core_axis_name="c", subcore_axis_name="s")
    run = functools.partial(
        pl.kernel,
        mesh=mesh,
        out_type=jax.ShapeDtypeStruct((NUM_HEADS * Q_LEN * K_LEN,), jnp.float32),
        scratch_types=[
            pltpu.VMEM((NSHIFT * DIAG,), jnp.float32),
            pltpu.SemaphoreType.DMA,
        ],
    )(_sc_body)
    out = run(diag8.reshape(NUM_HEADS * NSHIFT * DIAG))
    return out.reshape(1, NUM_HEADS, Q_LEN, K_LEN)
